# Initial kernel scaffold; baseline (speedup 1.0000x reference)
import jax
import jax.numpy as jnp
from jax.experimental import pallas as pl


def kernel(inputs, W0, W1, W2, W3, W4, W5):
    raise NotImplementedError("write your pallas kernel here")


# speedup vs baseline: 1.5437x; 1.5437x over previous
import functools

import jax
import jax.numpy as jnp
from jax import lax
from jax.experimental import pallas as pl
from jax.experimental.pallas import tpu as pltpu
from jax.experimental.pallas import tpu_sc as plsc

_VOCAB = 50304
_HIDDEN = 768
_N_EMB = 6
_TOTAL = 4 * 2048

_NC = 2
_NS = 16
_NW = _NC * _NS
_PER_W = _TOTAL // _NW
_CHUNK = 128
_NCHUNK = _PER_W // _CHUNK


@functools.cache
def _build():
    mesh = plsc.VectorSubcoreMesh(core_axis_name="c", subcore_axis_name="s")

    @functools.partial(
        pl.kernel,
        mesh=mesh,
        out_type=[jax.ShapeDtypeStruct((_TOTAL, _HIDDEN), jnp.float32)] * _N_EMB,
        scratch_types=[
            pltpu.VMEM((_PER_W,), jnp.int32),
            pltpu.VMEM((_CHUNK, _HIDDEN), jnp.float32),
            pltpu.SemaphoreType.DMA,
        ],
    )
    def _gather6(idx_hbm, w0, w1, w2, w3, w4, w5,
                 o0, o1, o2, o3, o4, o5, idx_v, rows_v, sem):
        wid = lax.axis_index("s") * _NC + lax.axis_index("c")
        base = wid * _PER_W
        pltpu.sync_copy(idx_hbm.at[pl.ds(base, _PER_W)], idx_v)
        for w, o in ((w0, o0), (w1, o1), (w2, o2),
                     (w3, o3), (w4, o4), (w5, o5)):
            for c in range(_NCHUNK):
                pltpu.async_copy(
                    w.at[idx_v.at[pl.ds(c * _CHUNK, _CHUNK)]], rows_v, sem
                ).wait()
                pltpu.sync_copy(rows_v, o.at[pl.ds(base + c * _CHUNK, _CHUNK)])

    return _gather6


def kernel(inputs, W0, W1, W2, W3, W4, W5):
    B, S = inputs.shape
    idx = inputs.reshape(-1).astype(jnp.int32)
    outs = _build()(idx, W0, W1, W2, W3, W4, W5)
    ve = [o.reshape(B, S, _HIDDEN) for o in outs]
    return tuple(ve + ve[::-1])


# speedup vs baseline: 1.5765x; 1.0212x over previous
import functools

import jax
import jax.numpy as jnp
from jax import lax
from jax.experimental import pallas as pl
from jax.experimental.pallas import tpu as pltpu
from jax.experimental.pallas import tpu_sc as plsc

_VOCAB = 50304
_HIDDEN = 768
_N_EMB = 6
_TOTAL = 4 * 2048

_NC = 2
_NS = 16
_NW = _NC * _NS
_PER_W = _TOTAL // _NW
_CHUNK = 64
_NCHUNK = _PER_W // _CHUNK


@functools.cache
def _build():
    mesh = plsc.VectorSubcoreMesh(core_axis_name="c", subcore_axis_name="s")

    @functools.partial(
        pl.kernel,
        mesh=mesh,
        out_type=[jax.ShapeDtypeStruct((_TOTAL, _HIDDEN), jnp.float32)] * _N_EMB,
        scratch_types=[
            pltpu.VMEM((_PER_W,), jnp.int32),
            pltpu.VMEM((_CHUNK, _HIDDEN), jnp.float32),
            pltpu.VMEM((_CHUNK, _HIDDEN), jnp.float32),
            pltpu.SemaphoreType.DMA,
            pltpu.SemaphoreType.DMA,
            pltpu.SemaphoreType.DMA,
            pltpu.SemaphoreType.DMA,
        ],
    )
    def _gather6(idx_hbm, w0, w1, w2, w3, w4, w5,
                 o0, o1, o2, o3, o4, o5,
                 idx_v, buf0, buf1, g0, g1, s0, s1):
        wid = lax.axis_index("s") * _NC + lax.axis_index("c")
        base = wid * _PER_W
        pltpu.sync_copy(idx_hbm.at[pl.ds(base, _PER_W)], idx_v)
        bufs, gsems, ssems = (buf0, buf1), (g0, g1), (s0, s1)
        steps = [(w, o, c)
                 for w, o in ((w0, o0), (w1, o1), (w2, o2),
                              (w3, o3), (w4, o4), (w5, o5))
                 for c in range(_NCHUNK)]

        def start_gather(i):
            w, _, c = steps[i]
            p = i % 2
            return pltpu.async_copy(
                w.at[idx_v.at[pl.ds(c * _CHUNK, _CHUNK)]], bufs[p], gsems[p])

        def start_write(i):
            _, o, c = steps[i]
            p = i % 2
            return pltpu.async_copy(
                bufs[p], o.at[pl.ds(base + c * _CHUNK, _CHUNK)], ssems[p])

        writes = [None, None]
        gather = start_gather(0)
        for i in range(len(steps)):
            p = i % 2
            nxt = None
            if i + 1 < len(steps):
                pn = (i + 1) % 2
                if writes[pn] is not None:
                    writes[pn].wait()
                nxt = start_gather(i + 1)
            gather.wait()
            writes[p] = start_write(i)
            gather = nxt
        for wr in writes:
            if wr is not None:
                wr.wait()

    return _gather6


def kernel(inputs, W0, W1, W2, W3, W4, W5):
    B, S = inputs.shape
    idx = inputs.reshape(-1).astype(jnp.int32)
    outs = _build()(idx, W0, W1, W2, W3, W4, W5)
    ve = [o.reshape(B, S, _HIDDEN) for o in outs]
    return tuple(ve + ve[::-1])


# speedup vs baseline: 1.8901x; 1.1990x over previous
import functools

import jax
import jax.numpy as jnp
from jax import lax
from jax.experimental import pallas as pl
from jax.experimental.pallas import tpu as pltpu
from jax.experimental.pallas import tpu_sc as plsc

_VOCAB = 50304
_HIDDEN = 768
_N_EMB = 6
_TOTAL = 4 * 2048

_NC = 2
_NS = 16
_NW = _NC * _NS
_PER_W = _TOTAL // _NW
_CHUNK = 64
_NCHUNK = _PER_W // _CHUNK


@functools.cache
def _build():
    mesh = plsc.VectorSubcoreMesh(core_axis_name="c", subcore_axis_name="s")

    @functools.partial(
        pl.kernel,
        mesh=mesh,
        out_type=[jax.ShapeDtypeStruct((_TOTAL, _HIDDEN), jnp.float32)]
        * (2 * _N_EMB),
        scratch_types=[
            pltpu.VMEM((_PER_W,), jnp.int32),
            pltpu.VMEM((_CHUNK, _HIDDEN), jnp.float32),
            pltpu.VMEM((_CHUNK, _HIDDEN), jnp.float32),
            pltpu.SemaphoreType.DMA,
            pltpu.SemaphoreType.DMA,
            pltpu.SemaphoreType.DMA,
            pltpu.SemaphoreType.DMA,
        ],
    )
    def _gather6(idx_hbm, w0, w1, w2, w3, w4, w5,
                 o0, o1, o2, o3, o4, o5, o6, o7, o8, o9, o10, o11,
                 idx_v, buf0, buf1, g0, g1, s0, s1):
        wid = lax.axis_index("s") * _NC + lax.axis_index("c")
        base = wid * _PER_W
        pltpu.sync_copy(idx_hbm.at[pl.ds(base, _PER_W)], idx_v)
        bufs, gsems, ssems = (buf0, buf1), (g0, g1), (s0, s1)
        outs = (o0, o1, o2, o3, o4, o5, o6, o7, o8, o9, o10, o11)
        ws = (w0, w1, w2, w3, w4, w5)
        steps = [(ws[t], outs[t], outs[11 - t], c)
                 for t in range(_N_EMB) for c in range(_NCHUNK)]

        def start_gather(i):
            w, _, _, c = steps[i]
            p = i % 2
            return pltpu.async_copy(
                w.at[idx_v.at[pl.ds(c * _CHUNK, _CHUNK)]], bufs[p], gsems[p])

        def start_writes(i):
            _, o_lo, o_hi, c = steps[i]
            p = i % 2
            dst = pl.ds(base + c * _CHUNK, _CHUNK)
            return [pltpu.async_copy(bufs[p], o_lo.at[dst], ssems[p]),
                    pltpu.async_copy(bufs[p], o_hi.at[dst], ssems[p])]

        writes = [[], []]
        gather = start_gather(0)
        for i in range(len(steps)):
            p = i % 2
            nxt = None
            if i + 1 < len(steps):
                pn = (i + 1) % 2
                for wr in writes[pn]:
                    wr.wait()
                nxt = start_gather(i + 1)
            gather.wait()
            writes[p] = start_writes(i)
            gather = nxt
        for wl in writes:
            for wr in wl:
                wr.wait()

    return _gather6


def kernel(inputs, W0, W1, W2, W3, W4, W5):
    B, S = inputs.shape
    idx = inputs.reshape(-1).astype(jnp.int32)
    outs = _build()(idx, W0, W1, W2, W3, W4, W5)
    return tuple(o.reshape(B, S, _HIDDEN) for o in outs)
